# baseline (speedup 1.0000x reference)
import functools

import jax
import jax.numpy as jnp
from jax import lax
from jax.experimental import pallas as pl
from jax.experimental.pallas import tpu as pltpu
from jax.experimental.pallas import tpu_sc as plsc

N = 10000
E = 320000
D_IN = 128
D_HID = 256
D_OUT = 128
HEADS = 4

NC = 2
NS = 16
NW = NC * NS
EPW = E // NW
B = 40
ITERS = EPW // B
B2 = 80
ITERS2 = EPW // B2
BG = 40
ITERS_G = EPW // BG
NP = 10240
RPS = NP // NS

DC = D_IN + 16

_mesh = plsc.VectorSubcoreMesh(
    core_axis_name="c", subcore_axis_name="s", num_cores=NC, num_subcores=NS)
_sc_params = pltpu.CompilerParams(use_tc_tiling_on_sc=False)
_sc_params_nl = pltpu.CompilerParams(use_tc_tiling_on_sc=False,
                                     needs_layout_passes=False)


def _pipeline(issue, compute, iters):
  issue(0, 0)

  def body(j, carry):
    jn = jnp.minimum(j + 1, iters - 1)

    @pl.when(j % 2 == 0)
    def _even():
      issue(jn, 1)
      compute(j, 0)

    @pl.when(j % 2 == 1)
    def _odd():
      issue(jn, 0)
      compute(j, 1)

    return carry

  lax.fori_loop(0, iters, body, 0)


def _make_sc_segsum(D, B, ITERS):
  @functools.partial(
      pl.kernel,
      out_type=jax.ShapeDtypeStruct((NC, NP, D), jnp.float32),
      mesh=_mesh,
      compiler_params=_sc_params,
      scratch_types=[
          pltpu.VMEM((ITERS, B), jnp.int32),
          pltpu.VMEM((ITERS, B), jnp.int32),
          pltpu.VMEM((2, B, D), jnp.float32),
          pltpu.VMEM_SHARED((NP, D), jnp.float32),
          pltpu.SemaphoreType.DMA,
          pltpu.SemaphoreType.DMA,
      ],
  )
  def k(table, src_blk, dst_blk, zeros, out, src_v, dst_v, rows2, acc,
        sem0, sem1):
    cid = lax.axis_index("c")
    sid = lax.axis_index("s")
    w = cid * NS + sid
    r0 = sid * RPS
    pltpu.sync_copy(zeros.at[pl.ds(r0, RPS)], acc.at[pl.ds(r0, RPS)])
    plsc.subcore_barrier()
    pltpu.sync_copy(src_blk.at[w], src_v)
    pltpu.sync_copy(dst_blk.at[w], dst_v)
    sems = (sem0, sem1)

    def issue(j, s):
      pltpu.async_copy(table.at[src_v.at[j]], rows2.at[s], sems[s])

    def compute(j, s):
      pltpu.make_async_copy(table.at[src_v.at[j]], rows2.at[s],
                            sems[s]).wait()
      pltpu.sync_copy(rows2.at[s], acc.at[dst_v.at[j]], add=True)

    _pipeline(issue, compute, ITERS)
    s_l = ITERS % 2
    pltpu.make_async_copy(table.at[src_v.at[0]], rows2.at[s_l],
                          sems[s_l]).wait()
    plsc.subcore_barrier()
    pltpu.sync_copy(acc.at[pl.ds(r0, RPS)], out.at[cid, pl.ds(r0, RPS)])

  return k


def _sc_attn_weights(es, ed, zeros16, src_blk, dst_blk):
  @functools.partial(
      pl.kernel,
      out_type=(
          jax.ShapeDtypeStruct((NC, NP, 16), jnp.float32),
          jax.ShapeDtypeStruct((NW, ITERS2, B2, 16), jnp.float32),
      ),
      mesh=_mesh,
      compiler_params=_sc_params,
      scratch_types=[
          pltpu.VMEM((ITERS2, B2), jnp.int32),
          pltpu.VMEM((ITERS2, B2), jnp.int32),
          pltpu.VMEM((2, B2, 16), jnp.float32),
          pltpu.VMEM((2, B2, 16), jnp.float32),
          pltpu.VMEM((B2, 16), jnp.float32),
          pltpu.VMEM_SHARED((NP, 16), jnp.float32),
          pltpu.SemaphoreType.DMA,
          pltpu.SemaphoreType.DMA,
          pltpu.SemaphoreType.DMA,
          pltpu.SemaphoreType.DMA,
      ],
  )
  def k(es_h, ed_h, z_h, src_blk_h, dst_blk_h, den_out, w_out,
        src_v, dst_v, es_r, ed_r, w_r, den, semA0, semA1, semB0, semB1):
    cid = lax.axis_index("c")
    sid = lax.axis_index("s")
    w = cid * NS + sid
    r0 = sid * RPS
    pltpu.sync_copy(z_h.at[pl.ds(r0, RPS)], den.at[pl.ds(r0, RPS)])
    plsc.subcore_barrier()
    pltpu.sync_copy(src_blk_h.at[w], src_v)
    pltpu.sync_copy(dst_blk_h.at[w], dst_v)
    semsA = (semA0, semA1)
    semsB = (semB0, semB1)

    def issue(j, s):
      pltpu.async_copy(es_h.at[src_v.at[j]], es_r.at[s], semsA[s])
      pltpu.async_copy(ed_h.at[dst_v.at[j]], ed_r.at[s], semsB[s])

    def compute(j, s):
      pltpu.make_async_copy(es_h.at[src_v.at[j]], es_r.at[s],
                            semsA[s]).wait()
      pltpu.make_async_copy(ed_h.at[dst_v.at[j]], ed_r.at[s],
                            semsB[s]).wait()

      def inner(i, c_):
        t = es_r[s, i, :] + ed_r[s, i, :]
        t = jnp.where(t >= 0.0, t, t * 0.2)
        w_r[i, :] = jnp.exp(t)
        return c_

      lax.fori_loop(0, B2, inner, 0)
      pltpu.sync_copy(w_r, den.at[dst_v.at[j]], add=True)
      pltpu.sync_copy(w_r, w_out.at[w, j])

    _pipeline(issue, compute, ITERS2)
    s_l = ITERS2 % 2
    pltpu.make_async_copy(es_h.at[src_v.at[0]], es_r.at[s_l],
                          semsA[s_l]).wait()
    pltpu.make_async_copy(ed_h.at[dst_v.at[0]], ed_r.at[s_l],
                          semsB[s_l]).wait()
    plsc.subcore_barrier()
    pltpu.sync_copy(den.at[pl.ds(r0, RPS)], den_out.at[cid, pl.ds(r0, RPS)])

  return k(es, ed, zeros16, src_blk, dst_blk)


def _sc_gat_aggregate(hg, invd, w_in, zeros128, src_blk, dst_blk):
  @functools.partial(
      pl.kernel,
      out_type=jax.ShapeDtypeStruct((NC, NP, D_OUT), jnp.float32),
      mesh=_mesh,
      compiler_params=_sc_params_nl,
      scratch_types=[
          pltpu.VMEM((ITERS_G, BG), jnp.int32),
          pltpu.VMEM((ITERS_G, BG), jnp.int32),
          pltpu.VMEM((2, BG, HEADS * D_OUT // 2), jnp.int32),
          pltpu.VMEM((2, BG, 16), jnp.float32),
          pltpu.VMEM((2, BG, 16), jnp.float32),
          pltpu.VMEM((BG, D_OUT), jnp.float32),
          pltpu.VMEM_SHARED((NP, D_OUT), jnp.float32),
          pltpu.SemaphoreType.DMA,
          pltpu.SemaphoreType.DMA,
          pltpu.SemaphoreType.DMA,
          pltpu.SemaphoreType.DMA,
          pltpu.SemaphoreType.DMA,
          pltpu.SemaphoreType.DMA,
      ],
  )
  def k(hg_h, invd_h, w_h, z_h, src_blk_h, dst_blk_h, out,
        src_v, dst_v, rows, invd_r, w_r, v_buf, acc,
        semA0, semA1, semB0, semB1, semC0, semC1):
    cid = lax.axis_index("c")
    sid = lax.axis_index("s")
    w = cid * NS + sid
    r0 = sid * RPS
    pltpu.sync_copy(z_h.at[pl.ds(r0, RPS)], acc.at[pl.ds(r0, RPS)])
    plsc.subcore_barrier()
    pltpu.sync_copy(src_blk_h.at[w], src_v)
    pltpu.sync_copy(dst_blk_h.at[w], dst_v)
    semsA = (semA0, semA1)
    semsB = (semB0, semB1)
    semsC = (semC0, semC1)

    def issue(j, s):
      pltpu.async_copy(hg_h.at[src_v.at[j]], rows.at[s], semsA[s])
      pltpu.async_copy(invd_h.at[dst_v.at[j]], invd_r.at[s], semsB[s])
      pltpu.async_copy(w_h.at[w, j], w_r.at[s], semsC[s])

    def waits(j, s):
      pltpu.make_async_copy(hg_h.at[src_v.at[j]], rows.at[s],
                            semsA[s]).wait()
      pltpu.make_async_copy(invd_h.at[dst_v.at[j]], invd_r.at[s],
                            semsB[s]).wait()
      pltpu.make_async_copy(w_h.at[w, j], w_r.at[s], semsC[s]).wait()

    def compute(j, s):
      waits(j, s)

      himask = jnp.int32(-65536)

      def inner(i, c_):
        av = w_r[s, i, :] * invd_r[s, i, :]
        al = (av[0], av[1], av[2], av[3])
        for q in range(4):
          vlo = None
          vhi = None
          for h in range(HEADS):
            xq = rows[s, i, pl.ds(h * 64 + q * 16, 16)]
            lo = plsc.bitcast(xq << 16, jnp.float32)
            hi = plsc.bitcast(xq & himask, jnp.float32)
            vlo = al[h] * lo if vlo is None else vlo + al[h] * lo
            vhi = al[h] * hi if vhi is None else vhi + al[h] * hi
          v_buf[i, pl.ds(q * 32, 16)] = vlo
          v_buf[i, pl.ds(q * 32 + 16, 16)] = vhi
        return c_

      lax.fori_loop(0, BG, inner, 0, unroll=4)
      pltpu.sync_copy(v_buf, acc.at[dst_v.at[j]], add=True)

    _pipeline(issue, compute, ITERS_G)
    waits(0, ITERS_G % 2)
    plsc.subcore_barrier()
    pltpu.sync_copy(acc.at[pl.ds(r0, RPS)], out.at[cid, pl.ds(r0, RPS)])

  return k(hg, invd, w_in, zeros128, src_blk, dst_blk)


NBLK = 10
RB = N // NBLK


def _tc1a(P, x, W_l1, W_r1, b1):
  def body(p_ref, x_ref, wl_ref, wr_ref, b_ref, hp_ref, st_ref):
    i = pl.program_id(0)
    s = p_ref[0] + p_ref[1]
    cnt = jnp.maximum(s[:, 128:129], 1.0)
    agg = s[:, :128] / cnt
    hp = (jnp.dot(agg, wl_ref[...], preferred_element_type=jnp.float32)
          + jnp.dot(x_ref[...], wr_ref[...], preferred_element_type=jnp.float32)
          + b_ref[...])
    hp_ref[...] = hp

    @pl.when(i == 0)
    def _():
      st_ref[...] = jnp.zeros_like(st_ref)

    st_ref[...] += jnp.concatenate(
        [jnp.sum(hp, axis=0, keepdims=True),
         jnp.sum(hp * hp, axis=0, keepdims=True)], axis=0)

  return pl.pallas_call(
      body,
      grid=(NBLK,),
      in_specs=[
          pl.BlockSpec((NC, RB, DC), lambda i: (0, i, 0)),
          pl.BlockSpec((RB, D_IN), lambda i: (i, 0)),
          pl.BlockSpec((D_IN, D_HID), lambda i: (0, 0)),
          pl.BlockSpec((D_IN, D_HID), lambda i: (0, 0)),
          pl.BlockSpec((1, D_HID), lambda i: (0, 0)),
      ],
      out_specs=[
          pl.BlockSpec((RB, D_HID), lambda i: (i, 0)),
          pl.BlockSpec((2, D_HID), lambda i: (0, 0)),
      ],
      out_shape=[
          jax.ShapeDtypeStruct((N, D_HID), jnp.float32),
          jax.ShapeDtypeStruct((2, D_HID), jnp.float32),
      ],
  )(P, x, W_l1, W_r1, b1)


def _tc1b(h_pre, stats, g1, be1, W_l2, W_r2):
  def body(hp_ref, st_ref, g_ref, be_ref, wl_ref, wr_ref, p2_ref, hr_ref):
    m = st_ref[0:1, :] * (1.0 / N)
    v = st_ref[1:2, :] * (1.0 / N) - m * m
    inv = g_ref[...] * lax.rsqrt(v + 1e-5)
    h = jnp.maximum((hp_ref[...] - m) * inv + be_ref[...], 0.0)
    p2_ref[...] = jnp.dot(h, wl_ref[...], preferred_element_type=jnp.float32)
    hr_ref[...] = jnp.dot(h, wr_ref[...], preferred_element_type=jnp.float32)

  return pl.pallas_call(
      body,
      grid=(NBLK,),
      in_specs=[
          pl.BlockSpec((RB, D_HID), lambda i: (i, 0)),
          pl.BlockSpec((2, D_HID), lambda i: (0, 0)),
          pl.BlockSpec((1, D_HID), lambda i: (0, 0)),
          pl.BlockSpec((1, D_HID), lambda i: (0, 0)),
          pl.BlockSpec((D_HID, D_OUT), lambda i: (0, 0)),
          pl.BlockSpec((D_HID, D_OUT), lambda i: (0, 0)),
      ],
      out_specs=[
          pl.BlockSpec((RB, D_OUT), lambda i: (i, 0)),
          pl.BlockSpec((RB, D_OUT), lambda i: (i, 0)),
      ],
      out_shape=[
          jax.ShapeDtypeStruct((N, D_OUT), jnp.float32),
          jax.ShapeDtypeStruct((N, D_OUT), jnp.float32),
      ],
  )(h_pre, stats, g1, be1, W_l2, W_r2)


def _tc2a(P, Q, hr2, b2):
  def body(p_ref, q_ref, hr_ref, b_ref, hp_ref, st_ref):
    i = pl.program_id(0)
    cnt = jnp.maximum(p_ref[0][:, 128:129] + p_ref[1][:, 128:129], 1.0)
    hp = (q_ref[0] + q_ref[1]) / cnt + hr_ref[...] + b_ref[...]
    hp_ref[...] = hp

    @pl.when(i == 0)
    def _():
      st_ref[...] = jnp.zeros_like(st_ref)

    st_ref[...] += jnp.concatenate(
        [jnp.sum(hp, axis=0, keepdims=True),
         jnp.sum(hp * hp, axis=0, keepdims=True)], axis=0)

  return pl.pallas_call(
      body,
      grid=(NBLK,),
      in_specs=[
          pl.BlockSpec((NC, RB, DC), lambda i: (0, i, 0)),
          pl.BlockSpec((NC, RB, D_OUT), lambda i: (0, i, 0)),
          pl.BlockSpec((RB, D_OUT), lambda i: (i, 0)),
          pl.BlockSpec((1, D_OUT), lambda i: (0, 0)),
      ],
      out_specs=[
          pl.BlockSpec((RB, D_OUT), lambda i: (i, 0)),
          pl.BlockSpec((2, D_OUT), lambda i: (0, 0)),
      ],
      out_shape=[
          jax.ShapeDtypeStruct((N, D_OUT), jnp.float32),
          jax.ShapeDtypeStruct((2, D_OUT), jnp.float32),
      ],
  )(P, Q, hr2, b2)


def _tc2b(h2_pre, stats, g2, be2, As, Ad):
  def body(hp_ref, st_ref, g_ref, be_ref, as_ref, ad_ref,
           h2_ref, es_ref, ed_ref):
    m = st_ref[0:1, :] * (1.0 / N)
    v = st_ref[1:2, :] * (1.0 / N) - m * m
    inv = g_ref[...] * lax.rsqrt(v + 1e-5)
    h2 = jnp.maximum((hp_ref[...] - m) * inv + be_ref[...], 0.0)
    h2_ref[...] = h2
    es_ref[...] = jnp.dot(h2, as_ref[...], preferred_element_type=jnp.float32)
    ed_ref[...] = jnp.dot(h2, ad_ref[...], preferred_element_type=jnp.float32)

  return pl.pallas_call(
      body,
      grid=(NBLK,),
      in_specs=[
          pl.BlockSpec((RB, D_OUT), lambda i: (i, 0)),
          pl.BlockSpec((2, D_OUT), lambda i: (0, 0)),
          pl.BlockSpec((1, D_OUT), lambda i: (0, 0)),
          pl.BlockSpec((1, D_OUT), lambda i: (0, 0)),
          pl.BlockSpec((D_OUT, 16), lambda i: (0, 0)),
          pl.BlockSpec((D_OUT, 16), lambda i: (0, 0)),
      ],
      out_specs=[
          pl.BlockSpec((RB, D_OUT), lambda i: (i, 0)),
          pl.BlockSpec((RB, 16), lambda i: (i, 0)),
          pl.BlockSpec((RB, 16), lambda i: (i, 0)),
      ],
      out_shape=[
          jax.ShapeDtypeStruct((N, D_OUT), jnp.float32),
          jax.ShapeDtypeStruct((N, 16), jnp.float32),
          jax.ShapeDtypeStruct((N, 16), jnp.float32),
      ],
  )(h2_pre, stats, g2, be2, As, Ad)


def _tc_hg(h2, W_gat):
  def body(h2_ref, wg_ref, hg_ref):
    hg = jnp.dot(h2_ref[...], wg_ref[...],
                 preferred_element_type=jnp.float32).astype(jnp.bfloat16)
    hg_ref[...] = hg

  return pl.pallas_call(
      body,
      grid=(NBLK,),
      in_specs=[
          pl.BlockSpec((RB, D_OUT), lambda i: (i, 0)),
          pl.BlockSpec((D_OUT, HEADS * D_OUT), lambda i: (0, 0)),
      ],
      out_specs=pl.BlockSpec((RB, HEADS * D_OUT), lambda i: (i, 0)),
      out_shape=jax.ShapeDtypeStruct((N, HEADS * D_OUT), jnp.bfloat16),
  )(h2, W_gat)


def _tc_invd(den):
  def body(d_ref, o_ref):
    o_ref[...] = 1.0 / jnp.maximum(d_ref[0] + d_ref[1], 1e-16)

  return pl.pallas_call(
      body,
      grid=(NBLK,),
      in_specs=[pl.BlockSpec((NC, RB, 16), lambda i: (0, i, 0))],
      out_specs=pl.BlockSpec((RB, 16), lambda i: (i, 0)),
      out_shape=jax.ShapeDtypeStruct((N, 16), jnp.float32),
  )(den)


def _tc3(A, b_gat):
  def body(a_ref, b_ref, o_ref):
    i = pl.program_id(0)

    @pl.when(i == 0)
    def _():
      o_ref[...] = jnp.zeros_like(o_ref)

    blk = jnp.maximum((a_ref[0] + a_ref[1]) * (1.0 / HEADS) + b_ref[...], 0.0)
    o_ref[...] += jnp.sum(blk, axis=0, keepdims=True) * (1.0 / N)

  return pl.pallas_call(
      body,
      grid=(NBLK,),
      in_specs=[
          pl.BlockSpec((NC, RB, D_OUT), lambda i: (0, i, 0)),
          pl.BlockSpec((1, D_OUT), lambda i: (0, 0)),
      ],
      out_specs=pl.BlockSpec((1, D_OUT), lambda i: (0, 0)),
      out_shape=jax.ShapeDtypeStruct((1, D_OUT), jnp.float32),
  )(A, b_gat)


_sc_segsum_xc = _make_sc_segsum(DC, B, ITERS)
_sc_segsum_p2 = _make_sc_segsum(D_OUT, B2, ITERS2)


def kernel(x, edge_index, W_l1, W_r1, b1, g1, be1, W_l2, W_r2, b2, g2, be2,
           W_gat, a_src, a_dst, b_gat):
  src_blk = edge_index[0].reshape(NW, ITERS, B)
  dst_blk = edge_index[1].reshape(NW, ITERS, B)
  src_b2 = edge_index[0].reshape(NW, ITERS2, B2)
  dst_b2 = edge_index[1].reshape(NW, ITERS2, B2)
  x_aug = jnp.concatenate([x, jnp.ones((N, 16), jnp.float32)], axis=1)
  zDC = jnp.zeros((NP, DC), jnp.float32)
  z128 = jnp.zeros((NP, D_OUT), jnp.float32)
  z16 = jnp.zeros((NP, 16), jnp.float32)
  Wg3 = W_gat.reshape(D_OUT, HEADS, D_OUT)
  As = jnp.einsum('khd,hd->kh', Wg3, a_src)
  Ad = jnp.einsum('khd,hd->kh', Wg3, a_dst)
  pad = jnp.zeros((D_OUT, 16 - HEADS), jnp.float32)
  As = jnp.concatenate([As, pad], axis=1)
  Ad = jnp.concatenate([Ad, pad], axis=1)

  P = _sc_segsum_xc(x_aug, src_blk, dst_blk, zDC)
  h_pre, st1 = _tc1a(P, x, W_l1, W_r1, b1.reshape(1, D_HID))
  p2, hr2 = _tc1b(h_pre, st1, g1.reshape(1, D_HID), be1.reshape(1, D_HID),
                  W_l2, W_r2)
  Q = _sc_segsum_p2(p2, src_b2, dst_b2, z128)
  h2_pre, st2 = _tc2a(P, Q, hr2, b2.reshape(1, D_OUT))
  h2, es, ed = _tc2b(h2_pre, st2, g2.reshape(1, D_OUT), be2.reshape(1, D_OUT),
                     As, Ad)
  den, w_e = _sc_attn_weights(es, ed, z16, src_b2, dst_b2)
  hg = _tc_hg(h2, W_gat)
  hg_i32 = lax.bitcast_convert_type(
      hg.reshape(N, HEADS * D_OUT // 2, 2), jnp.int32)
  invd = _tc_invd(den)
  src_g = edge_index[0].reshape(NW, ITERS_G, BG)
  dst_g = edge_index[1].reshape(NW, ITERS_G, BG)
  w_g = w_e.reshape(NW, ITERS_G, BG, 16)
  A = _sc_gat_aggregate(hg_i32, invd, w_g, z128, src_g, dst_g)
  bg_perm = b_gat.reshape(4, 16, 2).transpose(0, 2, 1).reshape(1, D_OUT)
  out_perm = _tc3(A, bg_perm)
  return out_perm.reshape(1, 4, 2, 16).transpose(0, 1, 3, 2).reshape(
      1, D_OUT)


# speedup vs baseline: 1.0463x; 1.0463x over previous
import functools

import jax
import jax.numpy as jnp
from jax import lax
from jax.experimental import pallas as pl
from jax.experimental.pallas import tpu as pltpu
from jax.experimental.pallas import tpu_sc as plsc

N = 10000
E = 320000
D_IN = 128
D_HID = 256
D_OUT = 128
HEADS = 4

NC = 2
NS = 16
NW = NC * NS
EPW = E // NW
B = 40
ITERS = EPW // B
B2 = 80
ITERS2 = EPW // B2
BG = 40
ITERS_G = EPW // BG
NP = 10240
RPS = NP // NS

DC = D_IN + 32

_mesh = plsc.VectorSubcoreMesh(
    core_axis_name="c", subcore_axis_name="s", num_cores=NC, num_subcores=NS)
_sc_params = pltpu.CompilerParams(use_tc_tiling_on_sc=False)
_sc_params_nl = pltpu.CompilerParams(use_tc_tiling_on_sc=False,
                                     needs_layout_passes=False)


def _pipeline(issue, compute, iters):
  issue(0, 0)

  def body(j, carry):
    jn = jnp.minimum(j + 1, iters - 1)

    @pl.when(j % 2 == 0)
    def _even():
      issue(jn, 1)
      compute(j, 0)

    @pl.when(j % 2 == 1)
    def _odd():
      issue(jn, 0)
      compute(j, 1)

    return carry

  lax.fori_loop(0, iters, body, 0)


def _make_sc_segsum(D, B, ITERS, dt):
  @functools.partial(
      pl.kernel,
      out_type=jax.ShapeDtypeStruct((NC, NP, D), dt),
      mesh=_mesh,
      compiler_params=_sc_params,
      scratch_types=[
          pltpu.VMEM((ITERS, B), jnp.int32),
          pltpu.VMEM((ITERS, B), jnp.int32),
          pltpu.VMEM((2, B, D), dt),
          pltpu.VMEM_SHARED((NP, D), dt),
          pltpu.SemaphoreType.DMA,
          pltpu.SemaphoreType.DMA,
      ],
  )
  def k(table, src_blk, dst_blk, zeros, out, src_v, dst_v, rows2, acc,
        sem0, sem1):
    cid = lax.axis_index("c")
    sid = lax.axis_index("s")
    w = cid * NS + sid
    r0 = sid * RPS
    pltpu.sync_copy(zeros.at[pl.ds(r0, RPS)], acc.at[pl.ds(r0, RPS)])
    plsc.subcore_barrier()
    pltpu.sync_copy(src_blk.at[w], src_v)
    pltpu.sync_copy(dst_blk.at[w], dst_v)
    sems = (sem0, sem1)

    def issue(j, s):
      pltpu.async_copy(table.at[src_v.at[j]], rows2.at[s], sems[s])

    def compute(j, s):
      pltpu.make_async_copy(table.at[src_v.at[j]], rows2.at[s],
                            sems[s]).wait()
      pltpu.sync_copy(rows2.at[s], acc.at[dst_v.at[j]], add=True)

    _pipeline(issue, compute, ITERS)
    s_l = ITERS % 2
    pltpu.make_async_copy(table.at[src_v.at[0]], rows2.at[s_l],
                          sems[s_l]).wait()
    plsc.subcore_barrier()
    pltpu.sync_copy(acc.at[pl.ds(r0, RPS)], out.at[cid, pl.ds(r0, RPS)])

  return k


def _sc_attn_weights(es, ed, zeros16, src_blk, dst_blk):
  @functools.partial(
      pl.kernel,
      out_type=(
          jax.ShapeDtypeStruct((NC, NP, 16), jnp.float32),
          jax.ShapeDtypeStruct((NW, ITERS2, B2, 16), jnp.float32),
      ),
      mesh=_mesh,
      compiler_params=_sc_params,
      scratch_types=[
          pltpu.VMEM((ITERS2, B2), jnp.int32),
          pltpu.VMEM((ITERS2, B2), jnp.int32),
          pltpu.VMEM((2, B2, 16), jnp.float32),
          pltpu.VMEM((2, B2, 16), jnp.float32),
          pltpu.VMEM((B2, 16), jnp.float32),
          pltpu.VMEM_SHARED((NP, 16), jnp.float32),
          pltpu.SemaphoreType.DMA,
          pltpu.SemaphoreType.DMA,
          pltpu.SemaphoreType.DMA,
          pltpu.SemaphoreType.DMA,
      ],
  )
  def k(es_h, ed_h, z_h, src_blk_h, dst_blk_h, den_out, w_out,
        src_v, dst_v, es_r, ed_r, w_r, den, semA0, semA1, semB0, semB1):
    cid = lax.axis_index("c")
    sid = lax.axis_index("s")
    w = cid * NS + sid
    r0 = sid * RPS
    pltpu.sync_copy(z_h.at[pl.ds(r0, RPS)], den.at[pl.ds(r0, RPS)])
    plsc.subcore_barrier()
    pltpu.sync_copy(src_blk_h.at[w], src_v)
    pltpu.sync_copy(dst_blk_h.at[w], dst_v)
    semsA = (semA0, semA1)
    semsB = (semB0, semB1)

    def issue(j, s):
      pltpu.async_copy(es_h.at[src_v.at[j]], es_r.at[s], semsA[s])
      pltpu.async_copy(ed_h.at[dst_v.at[j]], ed_r.at[s], semsB[s])

    def compute(j, s):
      pltpu.make_async_copy(es_h.at[src_v.at[j]], es_r.at[s],
                            semsA[s]).wait()
      pltpu.make_async_copy(ed_h.at[dst_v.at[j]], ed_r.at[s],
                            semsB[s]).wait()

      def inner(i, c_):
        t = es_r[s, i, :] + ed_r[s, i, :]
        t = jnp.where(t >= 0.0, t, t * 0.2)
        w_r[i, :] = jnp.exp(t)
        return c_

      lax.fori_loop(0, B2, inner, 0)
      pltpu.sync_copy(w_r, den.at[dst_v.at[j]], add=True)
      pltpu.sync_copy(w_r, w_out.at[w, j])

    _pipeline(issue, compute, ITERS2)
    s_l = ITERS2 % 2
    pltpu.make_async_copy(es_h.at[src_v.at[0]], es_r.at[s_l],
                          semsA[s_l]).wait()
    pltpu.make_async_copy(ed_h.at[dst_v.at[0]], ed_r.at[s_l],
                          semsB[s_l]).wait()
    plsc.subcore_barrier()
    pltpu.sync_copy(den.at[pl.ds(r0, RPS)], den_out.at[cid, pl.ds(r0, RPS)])

  return k(es, ed, zeros16, src_blk, dst_blk)


def _sc_gat_aggregate(hg, invd, w_in, zeros128, src_blk, dst_blk):
  @functools.partial(
      pl.kernel,
      out_type=jax.ShapeDtypeStruct((NC, NP, D_OUT), jnp.float32),
      mesh=_mesh,
      compiler_params=_sc_params_nl,
      scratch_types=[
          pltpu.VMEM((ITERS_G, BG), jnp.int32),
          pltpu.VMEM((ITERS_G, BG), jnp.int32),
          pltpu.VMEM((2, BG, HEADS * D_OUT // 2), jnp.int32),
          pltpu.VMEM((2, BG, 16), jnp.float32),
          pltpu.VMEM((2, BG, 16), jnp.float32),
          pltpu.VMEM((BG, D_OUT), jnp.float32),
          pltpu.VMEM_SHARED((NP, D_OUT), jnp.float32),
          pltpu.SemaphoreType.DMA,
          pltpu.SemaphoreType.DMA,
          pltpu.SemaphoreType.DMA,
          pltpu.SemaphoreType.DMA,
          pltpu.SemaphoreType.DMA,
          pltpu.SemaphoreType.DMA,
      ],
  )
  def k(hg_h, invd_h, w_h, z_h, src_blk_h, dst_blk_h, out,
        src_v, dst_v, rows, invd_r, w_r, v_buf, acc,
        semA0, semA1, semB0, semB1, semC0, semC1):
    cid = lax.axis_index("c")
    sid = lax.axis_index("s")
    w = cid * NS + sid
    r0 = sid * RPS
    pltpu.sync_copy(z_h.at[pl.ds(r0, RPS)], acc.at[pl.ds(r0, RPS)])
    plsc.subcore_barrier()
    pltpu.sync_copy(src_blk_h.at[w], src_v)
    pltpu.sync_copy(dst_blk_h.at[w], dst_v)
    semsA = (semA0, semA1)
    semsB = (semB0, semB1)
    semsC = (semC0, semC1)

    def issue(j, s):
      pltpu.async_copy(hg_h.at[src_v.at[j]], rows.at[s], semsA[s])
      pltpu.async_copy(invd_h.at[dst_v.at[j]], invd_r.at[s], semsB[s])
      pltpu.async_copy(w_h.at[w, j], w_r.at[s], semsC[s])

    def waits(j, s):
      pltpu.make_async_copy(hg_h.at[src_v.at[j]], rows.at[s],
                            semsA[s]).wait()
      pltpu.make_async_copy(invd_h.at[dst_v.at[j]], invd_r.at[s],
                            semsB[s]).wait()
      pltpu.make_async_copy(w_h.at[w, j], w_r.at[s], semsC[s]).wait()

    def compute(j, s):
      waits(j, s)

      himask = jnp.int32(-65536)

      def inner(i, c_):
        av = w_r[s, i, :] * invd_r[s, i, :]
        al = (av[0], av[1], av[2], av[3])
        for q in range(4):
          vlo = None
          vhi = None
          for h in range(HEADS):
            xq = rows[s, i, pl.ds(h * 64 + q * 16, 16)]
            lo = plsc.bitcast(xq << 16, jnp.float32)
            hi = plsc.bitcast(xq & himask, jnp.float32)
            vlo = al[h] * lo if vlo is None else vlo + al[h] * lo
            vhi = al[h] * hi if vhi is None else vhi + al[h] * hi
          v_buf[i, pl.ds(q * 32, 16)] = vlo
          v_buf[i, pl.ds(q * 32 + 16, 16)] = vhi
        return c_

      lax.fori_loop(0, BG, inner, 0, unroll=2)
      pltpu.sync_copy(v_buf, acc.at[dst_v.at[j]], add=True)

    _pipeline(issue, compute, ITERS_G)
    waits(0, ITERS_G % 2)
    plsc.subcore_barrier()
    pltpu.sync_copy(acc.at[pl.ds(r0, RPS)], out.at[cid, pl.ds(r0, RPS)])

  return k(hg, invd, w_in, zeros128, src_blk, dst_blk)


NBLK = 10
RB = N // NBLK


def _tc1a(P, x, W_l1, W_r1, b1):
  def body(p_ref, x_ref, wl_ref, wr_ref, b_ref, hp_ref, st_ref):
    i = pl.program_id(0)
    s = p_ref[0].astype(jnp.float32) + p_ref[1].astype(jnp.float32)
    cnt = jnp.maximum(s[:, 128:129], 1.0)
    agg = s[:, :128] / cnt
    hp = (jnp.dot(agg, wl_ref[...], preferred_element_type=jnp.float32)
          + jnp.dot(x_ref[...], wr_ref[...], preferred_element_type=jnp.float32)
          + b_ref[...])
    hp_ref[...] = hp

    @pl.when(i == 0)
    def _():
      st_ref[...] = jnp.zeros_like(st_ref)

    st_ref[...] += jnp.concatenate(
        [jnp.sum(hp, axis=0, keepdims=True),
         jnp.sum(hp * hp, axis=0, keepdims=True)], axis=0)

  return pl.pallas_call(
      body,
      grid=(NBLK,),
      in_specs=[
          pl.BlockSpec((NC, RB, DC), lambda i: (0, i, 0)),
          pl.BlockSpec((RB, D_IN), lambda i: (i, 0)),
          pl.BlockSpec((D_IN, D_HID), lambda i: (0, 0)),
          pl.BlockSpec((D_IN, D_HID), lambda i: (0, 0)),
          pl.BlockSpec((1, D_HID), lambda i: (0, 0)),
      ],
      out_specs=[
          pl.BlockSpec((RB, D_HID), lambda i: (i, 0)),
          pl.BlockSpec((2, D_HID), lambda i: (0, 0)),
      ],
      out_shape=[
          jax.ShapeDtypeStruct((N, D_HID), jnp.float32),
          jax.ShapeDtypeStruct((2, D_HID), jnp.float32),
      ],
  )(P, x, W_l1, W_r1, b1)


def _tc1b(h_pre, stats, g1, be1, W_l2, W_r2):
  def body(hp_ref, st_ref, g_ref, be_ref, wl_ref, wr_ref, p2_ref, hr_ref):
    m = st_ref[0:1, :] * (1.0 / N)
    v = st_ref[1:2, :] * (1.0 / N) - m * m
    inv = g_ref[...] * lax.rsqrt(v + 1e-5)
    h = jnp.maximum((hp_ref[...] - m) * inv + be_ref[...], 0.0)
    p2_ref[...] = jnp.dot(h, wl_ref[...],
                          preferred_element_type=jnp.float32).astype(
                              jnp.bfloat16)
    hr_ref[...] = jnp.dot(h, wr_ref[...], preferred_element_type=jnp.float32)

  return pl.pallas_call(
      body,
      grid=(NBLK,),
      in_specs=[
          pl.BlockSpec((RB, D_HID), lambda i: (i, 0)),
          pl.BlockSpec((2, D_HID), lambda i: (0, 0)),
          pl.BlockSpec((1, D_HID), lambda i: (0, 0)),
          pl.BlockSpec((1, D_HID), lambda i: (0, 0)),
          pl.BlockSpec((D_HID, D_OUT), lambda i: (0, 0)),
          pl.BlockSpec((D_HID, D_OUT), lambda i: (0, 0)),
      ],
      out_specs=[
          pl.BlockSpec((RB, D_OUT), lambda i: (i, 0)),
          pl.BlockSpec((RB, D_OUT), lambda i: (i, 0)),
      ],
      out_shape=[
          jax.ShapeDtypeStruct((N, D_OUT), jnp.bfloat16),
          jax.ShapeDtypeStruct((N, D_OUT), jnp.float32),
      ],
  )(h_pre, stats, g1, be1, W_l2, W_r2)


def _tc2a(P, Q, hr2, b2):
  def body(p_ref, q_ref, hr_ref, b_ref, hp_ref, st_ref):
    i = pl.program_id(0)
    cnt = jnp.maximum(p_ref[0][:, 128:129].astype(jnp.float32)
                      + p_ref[1][:, 128:129].astype(jnp.float32), 1.0)
    hp = ((q_ref[0].astype(jnp.float32) + q_ref[1].astype(jnp.float32)) / cnt
          + hr_ref[...] + b_ref[...])
    hp_ref[...] = hp

    @pl.when(i == 0)
    def _():
      st_ref[...] = jnp.zeros_like(st_ref)

    st_ref[...] += jnp.concatenate(
        [jnp.sum(hp, axis=0, keepdims=True),
         jnp.sum(hp * hp, axis=0, keepdims=True)], axis=0)

  return pl.pallas_call(
      body,
      grid=(NBLK,),
      in_specs=[
          pl.BlockSpec((NC, RB, DC), lambda i: (0, i, 0)),
          pl.BlockSpec((NC, RB, D_OUT), lambda i: (0, i, 0)),
          pl.BlockSpec((RB, D_OUT), lambda i: (i, 0)),
          pl.BlockSpec((1, D_OUT), lambda i: (0, 0)),
      ],
      out_specs=[
          pl.BlockSpec((RB, D_OUT), lambda i: (i, 0)),
          pl.BlockSpec((2, D_OUT), lambda i: (0, 0)),
      ],
      out_shape=[
          jax.ShapeDtypeStruct((N, D_OUT), jnp.float32),
          jax.ShapeDtypeStruct((2, D_OUT), jnp.float32),
      ],
  )(P, Q, hr2, b2)


def _tc2b(h2_pre, stats, g2, be2, As, Ad):
  def body(hp_ref, st_ref, g_ref, be_ref, as_ref, ad_ref,
           h2_ref, es_ref, ed_ref):
    m = st_ref[0:1, :] * (1.0 / N)
    v = st_ref[1:2, :] * (1.0 / N) - m * m
    inv = g_ref[...] * lax.rsqrt(v + 1e-5)
    h2 = jnp.maximum((hp_ref[...] - m) * inv + be_ref[...], 0.0)
    h2_ref[...] = h2
    es_ref[...] = jnp.dot(h2, as_ref[...], preferred_element_type=jnp.float32)
    ed_ref[...] = jnp.dot(h2, ad_ref[...], preferred_element_type=jnp.float32)

  return pl.pallas_call(
      body,
      grid=(NBLK,),
      in_specs=[
          pl.BlockSpec((RB, D_OUT), lambda i: (i, 0)),
          pl.BlockSpec((2, D_OUT), lambda i: (0, 0)),
          pl.BlockSpec((1, D_OUT), lambda i: (0, 0)),
          pl.BlockSpec((1, D_OUT), lambda i: (0, 0)),
          pl.BlockSpec((D_OUT, 16), lambda i: (0, 0)),
          pl.BlockSpec((D_OUT, 16), lambda i: (0, 0)),
      ],
      out_specs=[
          pl.BlockSpec((RB, D_OUT), lambda i: (i, 0)),
          pl.BlockSpec((RB, 16), lambda i: (i, 0)),
          pl.BlockSpec((RB, 16), lambda i: (i, 0)),
      ],
      out_shape=[
          jax.ShapeDtypeStruct((N, D_OUT), jnp.float32),
          jax.ShapeDtypeStruct((N, 16), jnp.float32),
          jax.ShapeDtypeStruct((N, 16), jnp.float32),
      ],
  )(h2_pre, stats, g2, be2, As, Ad)


def _tc_hg(h2, W_gat):
  def body(h2_ref, wg_ref, hg_ref):
    hg = jnp.dot(h2_ref[...], wg_ref[...],
                 preferred_element_type=jnp.float32).astype(jnp.bfloat16)
    hg_ref[...] = hg

  return pl.pallas_call(
      body,
      grid=(NBLK,),
      in_specs=[
          pl.BlockSpec((RB, D_OUT), lambda i: (i, 0)),
          pl.BlockSpec((D_OUT, HEADS * D_OUT), lambda i: (0, 0)),
      ],
      out_specs=pl.BlockSpec((RB, HEADS * D_OUT), lambda i: (i, 0)),
      out_shape=jax.ShapeDtypeStruct((N, HEADS * D_OUT), jnp.bfloat16),
  )(h2, W_gat)


def _tc_invd(den):
  def body(d_ref, o_ref):
    o_ref[...] = 1.0 / jnp.maximum(d_ref[0] + d_ref[1], 1e-16)

  return pl.pallas_call(
      body,
      grid=(NBLK,),
      in_specs=[pl.BlockSpec((NC, RB, 16), lambda i: (0, i, 0))],
      out_specs=pl.BlockSpec((RB, 16), lambda i: (i, 0)),
      out_shape=jax.ShapeDtypeStruct((N, 16), jnp.float32),
  )(den)


def _tc3(A, b_gat):
  def body(a_ref, b_ref, o_ref):
    i = pl.program_id(0)

    @pl.when(i == 0)
    def _():
      o_ref[...] = jnp.zeros_like(o_ref)

    blk = jnp.maximum((a_ref[0] + a_ref[1]) * (1.0 / HEADS) + b_ref[...], 0.0)
    o_ref[...] += jnp.sum(blk, axis=0, keepdims=True) * (1.0 / N)

  return pl.pallas_call(
      body,
      grid=(NBLK,),
      in_specs=[
          pl.BlockSpec((NC, RB, D_OUT), lambda i: (0, i, 0)),
          pl.BlockSpec((1, D_OUT), lambda i: (0, 0)),
      ],
      out_specs=pl.BlockSpec((1, D_OUT), lambda i: (0, 0)),
      out_shape=jax.ShapeDtypeStruct((1, D_OUT), jnp.float32),
  )(A, b_gat)


_sc_segsum_xc = _make_sc_segsum(DC, B2, ITERS2, jnp.bfloat16)
_sc_segsum_p2 = _make_sc_segsum(D_OUT, B2, ITERS2, jnp.bfloat16)


def kernel(x, edge_index, W_l1, W_r1, b1, g1, be1, W_l2, W_r2, b2, g2, be2,
           W_gat, a_src, a_dst, b_gat):
  src_blk = edge_index[0].reshape(NW, ITERS, B)
  dst_blk = edge_index[1].reshape(NW, ITERS, B)
  src_b2 = edge_index[0].reshape(NW, ITERS2, B2)
  dst_b2 = edge_index[1].reshape(NW, ITERS2, B2)
  x_aug = jnp.concatenate(
      [x.astype(jnp.bfloat16), jnp.ones((N, 32), jnp.bfloat16)], axis=1)
  zDC = jnp.zeros((NP, DC), jnp.bfloat16)
  z128b = jnp.zeros((NP, D_OUT), jnp.bfloat16)
  z128 = jnp.zeros((NP, D_OUT), jnp.float32)
  z16 = jnp.zeros((NP, 16), jnp.float32)
  Wg3 = W_gat.reshape(D_OUT, HEADS, D_OUT)
  As = jnp.einsum('khd,hd->kh', Wg3, a_src)
  Ad = jnp.einsum('khd,hd->kh', Wg3, a_dst)
  pad = jnp.zeros((D_OUT, 16 - HEADS), jnp.float32)
  As = jnp.concatenate([As, pad], axis=1)
  Ad = jnp.concatenate([Ad, pad], axis=1)

  P = _sc_segsum_xc(x_aug, src_b2, dst_b2, zDC)
  h_pre, st1 = _tc1a(P, x, W_l1, W_r1, b1.reshape(1, D_HID))
  p2, hr2 = _tc1b(h_pre, st1, g1.reshape(1, D_HID), be1.reshape(1, D_HID),
                  W_l2, W_r2)
  Q = _sc_segsum_p2(p2, src_b2, dst_b2, z128b)
  h2_pre, st2 = _tc2a(P, Q, hr2, b2.reshape(1, D_OUT))
  h2, es, ed = _tc2b(h2_pre, st2, g2.reshape(1, D_OUT), be2.reshape(1, D_OUT),
                     As, Ad)
  den, w_e = _sc_attn_weights(es, ed, z16, src_b2, dst_b2)
  hg = _tc_hg(h2, W_gat)
  hg_i32 = lax.bitcast_convert_type(
      hg.reshape(N, HEADS * D_OUT // 2, 2), jnp.int32)
  invd = _tc_invd(den)
  src_g = edge_index[0].reshape(NW, ITERS_G, BG)
  dst_g = edge_index[1].reshape(NW, ITERS_G, BG)
  w_g = w_e.reshape(NW, ITERS_G, BG, 16)
  A = _sc_gat_aggregate(hg_i32, invd, w_g, z128, src_g, dst_g)
  bg_perm = b_gat.reshape(4, 16, 2).transpose(0, 2, 1).reshape(1, D_OUT)
  out_perm = _tc3(A, bg_perm)
  return out_perm.reshape(1, 4, 2, 16).transpose(0, 1, 3, 2).reshape(
      1, D_OUT)
